# Initial kernel scaffold; baseline (speedup 1.0000x reference)
import functools

import jax
import jax.numpy as jnp
import numpy as np
from jax.experimental import pallas as pl
from jax.experimental.pallas import tpu as pltpu

_CELL = 30.0
_CUTOFF = 5.0


def _dense_kernel(xcol_ref, xrow_ref, z_ref, emb_ref, Wf_ref, W1_ref,
                  b1_ref, w2row_ref, b2_ref, out_ref, *, bi, ng, ntp):
    ti = pl.program_id(0)
    n = xrow_ref.shape[1]
    d = emb_ref.shape[1]
    i0 = ti * bi

    onehot = (z_ref[:, :] == jax.lax.broadcasted_iota(jnp.int32, (n, ntp), 1)
              ).astype(jnp.float32)
    h = jnp.dot(onehot, emb_ref[:, :], preferred_element_type=jnp.float32)

    dsq = jnp.zeros((bi, n), jnp.float32)
    for c in range(3):
        xi = xcol_ref[pl.ds(i0, bi), c:c + 1]
        xj = xrow_ref[c:c + 1, :]
        dm = xj - xi
        off = (dm < -0.5 * _CELL).astype(jnp.float32) - \
              (dm >= 0.5 * _CELL).astype(jnp.float32)
        dm = dm + off * _CELL
        dsq = dsq + dm * dm

    mask = (dsq < _CUTOFF * _CUTOFF) & (dsq != 0.0)
    r = jnp.where(mask, jnp.sqrt(dsq), jnp.float32(1e4))

    Wf = Wf_ref[:, :]
    step = jnp.float32(_CUTOFF / (ng - 1))

    def body(g, acc):
        cg = g.astype(jnp.float32) * step
        a = jnp.exp(-10.0 * (r - cg) ** 2)
        t = jnp.dot(a, h, preferred_element_type=jnp.float32)
        wg = jax.lax.dynamic_slice(Wf, (g, 0), (1, d))
        return acc + t * wg

    agg = jax.lax.fori_loop(0, ng, body, jnp.zeros((bi, d), jnp.float32))

    h_i = jax.lax.dynamic_slice(h, (i0, 0), (bi, d))
    hn = h_i + agg
    hidden = jnp.tanh(
        jnp.dot(hn, W1_ref[:, :], preferred_element_type=jnp.float32)
        + b1_ref[:, :])
    e_tile = jnp.sum(hidden * w2row_ref[:, :]) + bi * b2_ref[0, 0]

    @pl.when(ti == 0)
    def _():
        out_ref[:, :] = jnp.zeros_like(out_ref)

    out_ref[:, :] = out_ref[:, :] + e_tile[None, None]


@jax.jit
def kernel(q, z, emb, Wf, W1, b1, W2, b2):
    n = q.shape[0]
    d = emb.shape[1]
    ng = Wf.shape[0]
    ntypes = emb.shape[0]
    ntp = max(8, int(np.ceil(ntypes / 8)) * 8)

    bi = 128 if n % 128 == 0 else n
    grid = n // bi

    xcol = jnp.pad(q.astype(jnp.float32), ((0, 0), (0, 8 - 3)))
    xrow = jnp.pad(q.astype(jnp.float32).T, ((0, 8 - 3), (0, 0)))
    z2 = z.astype(jnp.int32).reshape(n, 1)
    embp = jnp.pad(emb, ((0, ntp - ntypes), (0, 0)))
    b1r = b1.reshape(1, d)
    w2row = W2.reshape(1, d)
    b2r = b2.reshape(1, 1)

    full = lambda shp: pl.BlockSpec(shp, lambda i: tuple(0 for _ in shp))
    out = pl.pallas_call(
        functools.partial(_dense_kernel, bi=bi, ng=ng, ntp=ntp),
        grid=(grid,),
        in_specs=[
            full((n, 8)), full((8, n)), full((n, 1)), full((ntp, d)),
            full((ng, d)), full((d, d)), full((1, d)), full((1, d)),
            full((1, 1)),
        ],
        out_specs=full((1, 1)),
        out_shape=jax.ShapeDtypeStruct((1, 1), jnp.float32),
        compiler_params=pltpu.CompilerParams(
            dimension_semantics=("arbitrary",)),
    )(xcol, xrow, z2, embp, Wf, W1, b1r, w2row, b2r)
    return out[0, 0]


# speedup vs baseline: 107.5549x; 107.5549x over previous
import functools

import jax
import jax.numpy as jnp
import numpy as np
from jax.experimental import pallas as pl
from jax.experimental.pallas import tpu as pltpu

_CELL = 30.0
_CUTOFF = 5.0


def _dense_kernel(xcol_ref, xrow_ref, z_ref, emb_ref, Wf_ref, W1_ref,
                  b1_ref, w2row_ref, b2_ref, out_ref, *, bi, ng, ntp):
    ti = pl.program_id(0)
    n = xrow_ref.shape[1]
    d = emb_ref.shape[1]
    i0 = ti * bi

    onehot = (z_ref[:, :] == jax.lax.broadcasted_iota(jnp.int32, (n, ntp), 1)
              ).astype(jnp.float32)
    h = jnp.dot(onehot, emb_ref[:, :], preferred_element_type=jnp.float32)

    dsq = jnp.zeros((bi, n), jnp.float32)
    for c in range(3):
        xi = xcol_ref[pl.ds(i0, bi), c:c + 1]
        xj = xrow_ref[c:c + 1, :]
        dm = xj - xi
        off = (dm < -0.5 * _CELL).astype(jnp.float32) - \
              (dm >= 0.5 * _CELL).astype(jnp.float32)
        dm = dm + off * _CELL
        dsq = dsq + dm * dm

    mask = (dsq < _CUTOFF * _CUTOFF) & (dsq != 0.0)
    r = jnp.where(mask, jnp.sqrt(dsq), jnp.float32(1e4))

    Wf = Wf_ref[:, :]
    step = jnp.float32(_CUTOFF / (ng - 1))

    grow = jax.lax.broadcasted_iota(jnp.int32, (ng, d), 0)

    def body(g, acc):
        cg = g.astype(jnp.float32) * step
        a = jnp.exp(-10.0 * (r - cg) ** 2)
        t = jnp.dot(a, h, preferred_element_type=jnp.float32)
        wg = jnp.sum(jnp.where(grow == g, Wf, 0.0), axis=0,
                     keepdims=True)
        return acc + t * wg

    agg = jax.lax.fori_loop(0, ng, body, jnp.zeros((bi, d), jnp.float32))

    onehot_i = (z_ref[pl.ds(i0, bi), :] ==
                jax.lax.broadcasted_iota(jnp.int32, (bi, ntp), 1)
                ).astype(jnp.float32)
    h_i = jnp.dot(onehot_i, emb_ref[:, :], preferred_element_type=jnp.float32)
    hn = h_i + agg
    hidden = jnp.tanh(
        jnp.dot(hn, W1_ref[:, :], preferred_element_type=jnp.float32)
        + b1_ref[:, :])
    e_tile = jnp.sum(hidden * w2row_ref[:, :]) + bi * b2_ref[0, 0]

    @pl.when(ti == 0)
    def _():
        out_ref[:, :] = jnp.zeros_like(out_ref)

    out_ref[:, :] = out_ref[:, :] + e_tile[None, None]


@jax.jit
def kernel(q, z, emb, Wf, W1, b1, W2, b2):
    n = q.shape[0]
    d = emb.shape[1]
    ng = Wf.shape[0]
    ntypes = emb.shape[0]
    ntp = max(8, int(np.ceil(ntypes / 8)) * 8)

    bi = 128 if n % 128 == 0 else n
    grid = n // bi

    xcol = jnp.pad(q.astype(jnp.float32), ((0, 0), (0, 8 - 3)))
    xrow = jnp.pad(q.astype(jnp.float32).T, ((0, 8 - 3), (0, 0)))
    z2 = z.astype(jnp.int32).reshape(n, 1)
    embp = jnp.pad(emb, ((0, ntp - ntypes), (0, 0)))
    b1r = b1.reshape(1, d)
    w2row = W2.reshape(1, d)
    b2r = b2.reshape(1, 1)

    full = lambda shp: pl.BlockSpec(shp, lambda i: tuple(0 for _ in shp))
    out = pl.pallas_call(
        functools.partial(_dense_kernel, bi=bi, ng=ng, ntp=ntp),
        grid=(grid,),
        in_specs=[
            full((n, 8)), full((8, n)), full((n, 1)), full((ntp, d)),
            full((ng, d)), full((d, d)), full((1, d)), full((1, d)),
            full((1, 1)),
        ],
        out_specs=full((1, 1)),
        out_shape=jax.ShapeDtypeStruct((1, 1), jnp.float32),
        compiler_params=pltpu.CompilerParams(
            dimension_semantics=("arbitrary",)),
    )(xcol, xrow, z2, embp, Wf, W1, b1r, w2row, b2r)
    return out[0, 0]
